# baseline (speedup 1.0000x reference)
import functools

import jax
import jax.numpy as jnp
from jax import lax
from jax.experimental import pallas as pl
from jax.experimental.pallas import tpu as pltpu
from jax.experimental.pallas import tpu_sc as plsc

N = 10000
E = 320000
IN_CH = 128
OUT_CH = 64

NC, NS = 2, 16
NW = NC * NS
NPAD = 10240
RPT = NPAD // NS
CHUNK = 128

E_H = 327680
CH_H = E_H // CHUNK
CPW_H = CH_H // NW

E_S = 344064
CH_S = E_S // CHUNK
CPW_S = CH_S // NW
PH = 2
CPW_P = CPW_S // PH

_MESH = plsc.VectorSubcoreMesh(core_axis_name="c", subcore_axis_name="s")


@functools.partial(
    pl.kernel,
    out_type=jax.ShapeDtypeStruct((NC, 1, NPAD), jnp.float32),
    mesh=_MESH,
    scratch_types=[
        pltpu.VMEM((CHUNK,), jnp.float32),
        pltpu.VMEM((CPW_H, CHUNK), jnp.int32),
        pltpu.VMEM_SHARED((NPAD,), jnp.float32),
    ],
)
def _hist(dst_hbm, zeros_hbm, deg_hbm, ones_v, idx_v, table):
    cid = lax.axis_index("c")
    sid = lax.axis_index("s")
    for i in range(CHUNK // 16):
        ones_v[pl.ds(i * 16, 16)] = jnp.full((16,), 1.0, jnp.float32)
    pltpu.sync_copy(dst_hbm.at[cid * NS + sid], idx_v)
    pltpu.sync_copy(zeros_hbm, table.at[pl.ds(sid * RPT, RPT)])
    plsc.subcore_barrier()

    def step(j, carry):
        pltpu.sync_copy(ones_v, table.at[idx_v.at[j]], add=True)
        return carry

    lax.fori_loop(0, CPW_H, step, 0)
    plsc.subcore_barrier()
    pltpu.sync_copy(table.at[pl.ds(sid * RPT, RPT)],
                    deg_hbm.at[cid, 0, pl.ds(sid * RPT, RPT)])


R_BLK = 512
GRID = NPAD // R_BLK


W_ROW = 128


def _dense_body(x_ref, w_ref, deg_ref, h_ref):
    xw = jnp.dot(x_ref[...], w_ref[...], preferred_element_type=jnp.float32)
    d = deg_ref[...]
    dis = lax.rsqrt(d[0] + d[1] + 1.0)
    h_ref[...] = jnp.concatenate(
        [xw * dis, jnp.zeros((R_BLK, W_ROW - OUT_CH), jnp.float32)], axis=1)


def _dense(x, w, deg3):
    return pl.pallas_call(
        _dense_body,
        grid=(GRID,),
        in_specs=[
            pl.BlockSpec((R_BLK, IN_CH), lambda i: (i, 0)),
            pl.BlockSpec((IN_CH, OUT_CH), lambda i: (0, 0)),
            pl.BlockSpec((NC, R_BLK, 1), lambda i: (0, i, 0)),
        ],
        out_specs=pl.BlockSpec((R_BLK, W_ROW), lambda i: (i, 0)),
        out_shape=jax.ShapeDtypeStruct((N, W_ROW), jnp.float32),
    )(x, w, deg3)


NB = 2


@functools.partial(
    pl.kernel,
    out_type=jax.ShapeDtypeStruct((NC, NPAD, W_ROW), jnp.float32),
    mesh=_MESH,
    scratch_types=[
        pltpu.VMEM((CPW_P, CHUNK), jnp.int32),
        pltpu.VMEM((CPW_P, CHUNK), jnp.int32),
        pltpu.VMEM((NB, CHUNK, W_ROW), jnp.float32),
        [pltpu.SemaphoreType.DMA] * NB,
        [pltpu.SemaphoreType.DMA] * NB,
        pltpu.VMEM_SHARED((NPAD, W_ROW), jnp.float32),
    ],
)
def _scatter(h_hbm, src_hbm, dst_hbm, zeros_hbm, agg_hbm,
             idxs_v, idxd_v, rows_v, sem_g, sem_s, acc):
    cid = lax.axis_index("c")
    sid = lax.axis_index("s")
    wid = cid * NS + sid
    pltpu.sync_copy(zeros_hbm, acc.at[pl.ds(sid * RPT, RPT)])
    plsc.subcore_barrier()

    def fire_gather(j, b):
        pltpu.async_copy(h_hbm.at[idxs_v.at[j]], rows_v.at[b], sem_g[b])

    def wait_gather(j, b):
        pltpu.make_async_copy(h_hbm.at[idxs_v.at[j]], rows_v.at[b],
                              sem_g[b]).wait()

    def drain_scatter(b):
        pltpu.make_async_copy(rows_v.at[b], acc.at[pl.ds(0, CHUNK)],
                              sem_s[b]).wait()

    for ph in range(PH):
        pltpu.sync_copy(src_hbm.at[wid, ph], idxs_v)
        pltpu.sync_copy(dst_hbm.at[wid, ph], idxd_v)

        for b in range(NB - 1):
            fire_gather(b, b)

        def group(g, carry):
            for b in range(NB):
                j = g * NB + b
                bprev = (b - 1) % NB

                @pl.when(jnp.logical_and(j >= 1, j + NB - 1 < CPW_P))
                def _():
                    drain_scatter(bprev)

                @pl.when(j + NB - 1 < CPW_P)
                def _():
                    fire_gather(j + NB - 1, bprev)

                wait_gather(j, b)
                pltpu.async_copy(rows_v.at[b], acc.at[idxd_v.at[j]],
                                 sem_s[b], add=True)
            return carry

        lax.fori_loop(0, CPW_P // NB, group, 0)
        for b in range(NB):
            drain_scatter((CPW_P - NB + b) % NB)

    plsc.subcore_barrier()
    pltpu.sync_copy(acc.at[pl.ds(sid * RPT, RPT)],
                    agg_hbm.at[cid, pl.ds(sid * RPT, RPT)])


def _finish_body(agg_ref, deg_ref, b_ref, a_ref, o_ref):
    m = agg_ref[...]
    s = m[0, :, :OUT_CH] + m[1, :, :OUT_CH]
    d = deg_ref[...]
    dis = lax.rsqrt(d[0] + d[1] + 1.0)
    o = s * dis + b_ref[...]
    o_ref[...] = jnp.where(o > 0, o, a_ref[...] * o)


def _finish(agg, deg3, b2, a2):
    return pl.pallas_call(
        _finish_body,
        grid=(GRID,),
        in_specs=[
            pl.BlockSpec((NC, R_BLK, W_ROW), lambda i: (0, i, 0)),
            pl.BlockSpec((NC, R_BLK, 1), lambda i: (0, i, 0)),
            pl.BlockSpec((1, OUT_CH), lambda i: (0, 0)),
            pl.BlockSpec((1, OUT_CH), lambda i: (0, 0)),
        ],
        out_specs=pl.BlockSpec((R_BLK, OUT_CH), lambda i: (i, 0)),
        out_shape=jax.ShapeDtypeStruct((N, OUT_CH), jnp.float32),
    )(agg, deg3, b2, a2)


def kernel(x, edge_index, W, b, prelu_a):
    src = edge_index[0].astype(jnp.int32)
    dst = edge_index[1].astype(jnp.int32)

    dst_h = jnp.concatenate(
        [dst, jnp.full((E_H - E,), N, jnp.int32)]).reshape(NW, CPW_H, CHUNK)
    zeros1 = jnp.zeros((RPT,), jnp.float32)
    deg = _hist(dst_h, zeros1)
    deg3 = deg.reshape(NC, NPAD, 1)

    h = _dense(x, W, deg3)

    loop = jnp.arange(N, dtype=jnp.int32)
    pad = E_S - E - N
    trash = N + jnp.arange(pad, dtype=jnp.int32) % (NPAD - N)
    srcp = jnp.concatenate(
        [src, loop, jnp.zeros((pad,), jnp.int32)]).reshape(NW, PH, CPW_P, CHUNK)
    dstp = jnp.concatenate(
        [dst, loop, trash]).reshape(NW, PH, CPW_P, CHUNK)
    zeros2 = jnp.zeros((RPT, W_ROW), jnp.float32)
    agg = _scatter(h, srcp, dstp, zeros2)

    return _finish(agg, deg3, b.reshape(1, OUT_CH), prelu_a.reshape(1, OUT_CH))


# speedup vs baseline: 2.1416x; 2.1416x over previous
import functools

import jax
import jax.numpy as jnp
from jax import lax
from jax.experimental import pallas as pl
from jax.experimental.pallas import tpu as pltpu
from jax.experimental.pallas import tpu_sc as plsc

N = 10000
E = 320000
IN_CH = 128
OUT_CH = 64

NC, NS = 2, 16
NW = NC * NS
NPAD = 10240
RPT = NPAD // NS
CHUNK = 128

E_H = 323584
CH_H = E_H // CHUNK
CPW_H = CH_H // NW

E_S = 331776
CH_S = E_S // CHUNK
CPW_S = CH_S // NW

_MESH = plsc.VectorSubcoreMesh(core_axis_name="c", subcore_axis_name="s")


@functools.partial(
    pl.kernel,
    out_type=jax.ShapeDtypeStruct((NC, 1, NPAD), jnp.float32),
    mesh=_MESH,
    scratch_types=[
        pltpu.VMEM((CHUNK,), jnp.float32),
        pltpu.VMEM((CPW_H, CHUNK), jnp.int32),
        pltpu.VMEM_SHARED((NPAD,), jnp.float32),
    ],
)
def _hist(dst_hbm, zeros_hbm, deg_hbm, ones_v, idx_v, table):
    cid = lax.axis_index("c")
    sid = lax.axis_index("s")
    for i in range(CHUNK // 16):
        ones_v[pl.ds(i * 16, 16)] = jnp.full((16,), 1.0, jnp.float32)
    pltpu.sync_copy(dst_hbm.at[cid * NS + sid], idx_v)
    pltpu.sync_copy(zeros_hbm, table.at[pl.ds(sid * RPT, RPT)])
    plsc.subcore_barrier()

    def step(j, carry):
        pltpu.sync_copy(ones_v, table.at[idx_v.at[j]], add=True)
        return carry

    lax.fori_loop(0, CPW_H, step, 0)
    plsc.subcore_barrier()
    pltpu.sync_copy(table.at[pl.ds(sid * RPT, RPT)],
                    deg_hbm.at[cid, 0, pl.ds(sid * RPT, RPT)])


R_BLK = 512
GRID = NPAD // R_BLK


W_ROW = 128


def _dense_body(x_ref, w_ref, deg_ref, h_ref):
    xw = jnp.dot(x_ref[...], w_ref[...], preferred_element_type=jnp.float32)
    d = deg_ref[...]
    dis = lax.rsqrt(d[0] + d[1] + 1.0)
    h_ref[...] = jnp.concatenate(
        [xw * dis, jnp.zeros((R_BLK, W_ROW - OUT_CH), jnp.float32)], axis=1)


def _dense(x, w, deg3):
    return pl.pallas_call(
        _dense_body,
        grid=(GRID,),
        in_specs=[
            pl.BlockSpec((R_BLK, IN_CH), lambda i: (i, 0)),
            pl.BlockSpec((IN_CH, OUT_CH), lambda i: (0, 0)),
            pl.BlockSpec((NC, R_BLK, 1), lambda i: (0, i, 0)),
        ],
        out_specs=pl.BlockSpec((R_BLK, W_ROW), lambda i: (i, 0)),
        out_shape=jax.ShapeDtypeStruct((N, W_ROW), jnp.float32),
    )(x, w, deg3)


@functools.partial(
    pl.kernel,
    out_type=jax.ShapeDtypeStruct((NC, NPAD, W_ROW), jnp.float32),
    mesh=_MESH,
    scratch_types=[
        pltpu.VMEM((CPW_S, CHUNK), jnp.int32),
        pltpu.VMEM((CPW_S, CHUNK), jnp.int32),
        pltpu.VMEM((CHUNK, W_ROW), jnp.float32),
        pltpu.SemaphoreType.DMA,
        pltpu.VMEM_SHARED((NPAD, W_ROW), jnp.float32),
    ],
)
def _scatter(h_hbm, src_hbm, dst_hbm, zeros_hbm, agg_hbm,
             idxs_v, idxd_v, rows_v, sem, acc):
    cid = lax.axis_index("c")
    sid = lax.axis_index("s")
    wid = cid * NS + sid
    pltpu.sync_copy(src_hbm.at[wid], idxs_v)
    pltpu.sync_copy(dst_hbm.at[wid], idxd_v)
    pltpu.sync_copy(zeros_hbm, acc.at[pl.ds(sid * RPT, RPT)])
    plsc.subcore_barrier()

    def step(j, carry):
        pltpu.async_copy(h_hbm.at[idxs_v.at[j]], rows_v, sem).wait()
        pltpu.sync_copy(rows_v, acc.at[idxd_v.at[j]], add=True)
        return carry

    lax.fori_loop(0, CPW_S, step, 0)

    plsc.subcore_barrier()
    pltpu.sync_copy(acc.at[pl.ds(sid * RPT, RPT)],
                    agg_hbm.at[cid, pl.ds(sid * RPT, RPT)])


def _finish_body(agg_ref, deg_ref, b_ref, a_ref, o_ref):
    m = agg_ref[...]
    s = m[0, :, :OUT_CH] + m[1, :, :OUT_CH]
    d = deg_ref[...]
    dis = lax.rsqrt(d[0] + d[1] + 1.0)
    o = s * dis + b_ref[...]
    o_ref[...] = jnp.where(o > 0, o, a_ref[...] * o)


def _finish(agg, deg3, b2, a2):
    return pl.pallas_call(
        _finish_body,
        grid=(GRID,),
        in_specs=[
            pl.BlockSpec((NC, R_BLK, W_ROW), lambda i: (0, i, 0)),
            pl.BlockSpec((NC, R_BLK, 1), lambda i: (0, i, 0)),
            pl.BlockSpec((1, OUT_CH), lambda i: (0, 0)),
            pl.BlockSpec((1, OUT_CH), lambda i: (0, 0)),
        ],
        out_specs=pl.BlockSpec((R_BLK, OUT_CH), lambda i: (i, 0)),
        out_shape=jax.ShapeDtypeStruct((N, OUT_CH), jnp.float32),
    )(agg, deg3, b2, a2)


def kernel(x, edge_index, W, b, prelu_a):
    src = edge_index[0].astype(jnp.int32)
    dst = edge_index[1].astype(jnp.int32)

    dst_h = jnp.concatenate(
        [dst, jnp.full((E_H - E,), N, jnp.int32)]).reshape(NW, CPW_H, CHUNK)
    zeros1 = jnp.zeros((RPT,), jnp.float32)
    deg = _hist(dst_h, zeros1)
    deg3 = deg.reshape(NC, NPAD, 1)

    h = _dense(x, W, deg3)

    loop = jnp.arange(N, dtype=jnp.int32)
    pad = E_S - E - N
    trash = N + jnp.arange(pad, dtype=jnp.int32) % (NPAD - N)
    srcp = jnp.concatenate(
        [src, loop, jnp.zeros((pad,), jnp.int32)]).reshape(NW, CPW_S, CHUNK)
    dstp = jnp.concatenate(
        [dst, loop, trash]).reshape(NW, CPW_S, CHUNK)
    zeros2 = jnp.zeros((RPT, W_ROW), jnp.float32)
    agg = _scatter(h, srcp, dstp, zeros2)

    return _finish(agg, deg3, b.reshape(1, OUT_CH), prelu_a.reshape(1, OUT_CH))
